# baseline (speedup 1.0000x reference)
import functools

import jax
import jax.numpy as jnp
from jax import lax
from jax.experimental import pallas as pl
from jax.experimental.pallas import tpu as pltpu
from jax.experimental.pallas import tpu_sc as plsc

_HBLK_POOL = 96
_HBLK_MIX = 96
_LANES = 16


def _pool_body(x_ref, w1_ref, b1_ref, w2p_ref, b2p_ref, logits_ref, acc_ref,
               *, n_spatial, n_experts):
    t = pl.program_id(0)

    @pl.when(t == 0)
    def _init():
        acc_ref[...] = jnp.zeros_like(acc_ref)

    acc_ref[...] += jnp.sum(x_ref[...], axis=(2, 3))

    @pl.when(t == pl.num_programs(0) - 1)
    def _epilogue():
        B = acc_ref.shape[0]
        pooled = acc_ref[...] * (1.0 / n_spatial)
        h = jax.nn.gelu(jnp.dot(pooled, w1_ref[...],
                                preferred_element_type=jnp.float32)
                        + b1_ref[...][None, :])
        logits = (jnp.dot(h, w2p_ref[...],
                          preferred_element_type=jnp.float32)
                  + b2p_ref[...][None, :])
        pad = lax.broadcasted_iota(jnp.int32, (B, 1, _LANES), 2) >= n_experts
        logits_ref[...] = jnp.broadcast_to(
            jnp.where(pad, -1e30, logits[:, None, :]),
            logits_ref.shape)


def _sc_route_body(logits_hbm, wexp_hbm, bexp_hbm, wtot_hbm, bcomb_hbm,
                   lref, idx_ref, rows_ref, acc2d_ref, brows_ref, bacc_ref,
                   sem, *, top_k, n_rows_per_sub, n_sub):
    b = lax.axis_index("c")
    sid = lax.axis_index("s")
    C = wtot_hbm.shape[1]
    n_i = C // _LANES
    n_active = C // n_rows_per_sub

    pltpu.sync_copy(logits_hbm.at[b], lref)
    l = lref[0, :]
    lane = lax.broadcasted_iota(jnp.int32, (_LANES,), 0)

    beats = jnp.zeros((_LANES,), jnp.int32)
    for k in range(1, _LANES):
        rk = (lane + k) % _LANES
        lv = l[rk]
        tie = jnp.where(lv == l, jnp.where(rk < lane, 1, 0), 0)
        beats += jnp.where(lv > l, 1, tie)
    sel = beats < top_k

    m = jnp.where(sel, l, -1e30)
    for sh in (1, 2, 4, 8):
        m = jnp.maximum(m, m[lane ^ sh])
    ex = jnp.where(sel, jnp.exp(l - m), 0.0)
    s = ex
    for sh in (1, 2, 4, 8):
        s = s + s[lane ^ sh]
    w = ex / s

    idxv = jnp.zeros((_LANES,), jnp.int32)
    wk = []
    for j in range(top_k):
        mj = beats == j
        ij = jnp.where(mj, lane, 0)
        wj = jnp.where(mj, w, 0.0)
        for sh in (1, 2, 4, 8):
            ij = ij + ij[lane ^ sh]
            wj = wj + wj[lane ^ sh]
        idxv = jnp.where(lane == j, ij, idxv)
        wk.append(wj)

    idx_ref[...] = idxv
    idx4 = idx_ref.at[pl.ds(0, top_k)]
    pltpu.async_copy(wexp_hbm.at[idx4], rows_ref, sem).wait()

    @pl.when(sid < n_active)
    def _combine():
        for r in range(n_rows_per_sub):
            o = sid * n_rows_per_sub + r
            for i in range(n_i):
                v = jnp.zeros((_LANES,), jnp.float32)
                for k in range(top_k):
                    v += wk[k] * rows_ref[k, pl.ds(o * C + i * _LANES,
                                                   _LANES)]
                v += jnp.where(i * _LANES + lane == o, 1.0, 0.0)
                acc2d_ref[r, pl.ds(i * _LANES, _LANES)] = v
        pltpu.sync_copy(acc2d_ref,
                        wtot_hbm.at[b, pl.ds(sid * n_rows_per_sub,
                                             n_rows_per_sub), :])

    @pl.when(sid == n_sub - 1)
    def _bias():
        pltpu.async_copy(bexp_hbm.at[idx_ref.at[pl.ds(0, top_k)]],
                         brows_ref, sem).wait()
        for i in range(n_i):
            v = jnp.zeros((_LANES,), jnp.float32)
            for k in range(top_k):
                v += wk[k] * brows_ref[k, pl.ds(i * _LANES, _LANES)]
            bacc_ref[0, pl.ds(i * _LANES, _LANES)] = v
        pltpu.sync_copy(bacc_ref, bcomb_hbm.at[b])


def _mix_body(x_ref, wtot_ref, bcomb_ref, out_ref, *, hblk):
    w = wtot_ref[0]
    bc = bcomb_ref[0]
    for h in range(hblk):
        out_ref[0, :, h, :] = (
            jnp.dot(w, x_ref[0, :, h, :], preferred_element_type=jnp.float32)
            + bc)


@jax.jit
def kernel(x, W1, b1, W2, b2, Wexp, bexp):
    B, C, H, Wd = x.shape
    E = Wexp.shape[0]
    top_k = 4
    HW = H * Wd

    W2p = jnp.zeros((C // 4, _LANES), jnp.float32).at[:, :E].set(W2)
    b2p = jnp.zeros((_LANES,), jnp.float32).at[:E].set(b2)
    T1 = H // _HBLK_POOL
    logits = pl.pallas_call(
        functools.partial(_pool_body, n_spatial=HW, n_experts=E),
        grid=(T1,),
        in_specs=[
            pl.BlockSpec((B, C, _HBLK_POOL, Wd), lambda t: (0, 0, t, 0)),
            pl.BlockSpec((C, C // 4), lambda t: (0, 0)),
            pl.BlockSpec((C // 4,), lambda t: (0,)),
            pl.BlockSpec((C // 4, _LANES), lambda t: (0, 0)),
            pl.BlockSpec((_LANES,), lambda t: (0,)),
        ],
        out_specs=pl.BlockSpec((B, 8, _LANES), lambda t: (0, 0, 0)),
        out_shape=jax.ShapeDtypeStruct((B, 8, _LANES), jnp.float32),
        scratch_shapes=[pltpu.VMEM((B, C), jnp.float32)],
    )(x, W1, b1, W2p, b2p)

    n_sub = plsc.get_sparse_core_info().num_subcores
    n_rows = 8
    mesh = plsc.VectorSubcoreMesh(core_axis_name="c", subcore_axis_name="s")
    wtot, bcomb8 = pl.kernel(
        functools.partial(_sc_route_body, top_k=top_k,
                          n_rows_per_sub=n_rows, n_sub=n_sub),
        out_type=[
            jax.ShapeDtypeStruct((B, C, C), jnp.float32),
            jax.ShapeDtypeStruct((B, 8, 128), jnp.float32),
        ],
        mesh=mesh,
        scratch_types=[
            pltpu.VMEM((8, _LANES), jnp.float32),
            pltpu.VMEM((_LANES,), jnp.int32),
            pltpu.VMEM((top_k, C * C), jnp.float32),
            pltpu.VMEM((n_rows, C), jnp.float32),
            pltpu.VMEM((top_k, 128), jnp.float32),
            pltpu.VMEM((8, 128), jnp.float32),
            pltpu.SemaphoreType.DMA,
        ],
    )(logits, Wexp.reshape(E, C * C),
      jnp.zeros((E, 128), jnp.float32).at[:, :C].set(bexp))

    bcomb3 = bcomb8[:, 0, :C].reshape(B, C, 1)

    T2 = H // _HBLK_MIX
    out = pl.pallas_call(
        functools.partial(_mix_body, hblk=_HBLK_MIX),
        grid=(B, T2),
        in_specs=[
            pl.BlockSpec((1, C, _HBLK_MIX, Wd), lambda b, t: (b, 0, t, 0)),
            pl.BlockSpec((1, C, C), lambda b, t: (b, 0, 0)),
            pl.BlockSpec((1, C, 1), lambda b, t: (b, 0, 0)),
        ],
        out_specs=pl.BlockSpec((1, C, _HBLK_MIX, Wd), lambda b, t: (b, 0, t, 0)),
        out_shape=jax.ShapeDtypeStruct((B, C, H, Wd), jnp.float32),
    )(x, wtot, bcomb3)

    return out
